# baseline (speedup 1.0000x reference)
import jax
import jax.numpy as jnp
from jax.experimental import pallas as pl

_N = 10000
_C = 128
_BLK = 5000
_P = 3 * _C + 8


def _body(x_ref, p_ref, o_ref):
    xb = x_ref[...]
    wcat = jnp.concatenate([p_ref[0:_C, :], p_ref[_C:2 * _C, :]], axis=1)
    t = jnp.dot(xb, wcat, preferred_element_type=jnp.float32)
    a = t[:, :_C] + p_ref[3 * _C, :]
    b = t[:, _C:] + p_ref[3 * _C + 1, :]
    hpre = jax.nn.sigmoid(-a) * jnp.tanh(b)
    h = jnp.where(hpre > 0, hpre, jnp.exp(hpre) - 1.0)
    o_ref[...] = (
        jnp.dot(h, p_ref[2 * _C:3 * _C, :], preferred_element_type=jnp.float32)
        + p_ref[3 * _C + 2, :]
    )


def kernel(x, edge_index, edge_weight, W_xz, b_xz, W_hz, b_hz, W_xr, b_xr,
           W_hr, b_hr, W_xh, b_xh, W_hh, b_hh, W_lin, b_lin):
    bias_block = jnp.stack(
        [b_xz + b_hz, b_xh + b_hh, b_lin,
         b_lin, b_lin, b_lin, b_lin, b_lin])
    params = jnp.concatenate([W_xz, W_xh, W_lin, bias_block], axis=0)
    grid = (_N // _BLK,)
    return pl.pallas_call(
        _body,
        grid=grid,
        in_specs=[
            pl.BlockSpec((_BLK, _C), lambda i: (i, 0)),
            pl.BlockSpec((_P, _C), lambda i: (0, 0)),
        ],
        out_specs=pl.BlockSpec((_BLK, _C), lambda i: (i, 0)),
        out_shape=jax.ShapeDtypeStruct((_N, _C), jnp.float32),
    )(x, params)


# speedup vs baseline: 1.7237x; 1.7237x over previous
import jax
import jax.numpy as jnp
from jax.experimental import pallas as pl
from jax.experimental.pallas import tpu as pltpu

_N = 10000
_C = 128
_BLK = 5000


def _body(x_ref, wxz_ref, wxh_ref, wlin_ref,
          bxz_ref, bhz_ref, bxh_ref, bhh_ref, blin_ref, o_ref,
          w_s, b_s, sems):
    @pl.when(pl.program_id(0) == 0)
    def _load_params():
        cps = [
            pltpu.make_async_copy(wxz_ref, w_s.at[0], sems.at[0]),
            pltpu.make_async_copy(wxh_ref, w_s.at[1], sems.at[1]),
            pltpu.make_async_copy(wlin_ref, w_s.at[2], sems.at[2]),
            pltpu.make_async_copy(bxz_ref, b_s.at[0], sems.at[3]),
            pltpu.make_async_copy(bhz_ref, b_s.at[1], sems.at[4]),
            pltpu.make_async_copy(bxh_ref, b_s.at[2], sems.at[5]),
            pltpu.make_async_copy(bhh_ref, b_s.at[3], sems.at[6]),
            pltpu.make_async_copy(blin_ref, b_s.at[4], sems.at[7]),
        ]
        for cp in cps:
            cp.start()
        for cp in cps:
            cp.wait()

    xb = x_ref[...]
    wcat = jnp.concatenate([w_s[0], w_s[1]], axis=1)
    t = jnp.dot(xb, wcat, preferred_element_type=jnp.float32)
    a = t[:, :_C] + (b_s[0] + b_s[1])
    b = t[:, _C:] + (b_s[2] + b_s[3])
    hpre = jax.nn.sigmoid(-a) * jnp.tanh(b)
    h = jnp.where(hpre > 0, hpre, jnp.exp(hpre) - 1.0)
    o_ref[...] = (
        jnp.dot(h, w_s[2], preferred_element_type=jnp.float32)
        + b_s[4]
    )


def kernel(x, edge_index, edge_weight, W_xz, b_xz, W_hz, b_hz, W_xr, b_xr,
           W_hr, b_hr, W_xh, b_xh, W_hh, b_hh, W_lin, b_lin):
    grid = (_N // _BLK,)
    anyspec = pl.BlockSpec(memory_space=pltpu.MemorySpace.HBM)
    return pl.pallas_call(
        _body,
        grid=grid,
        in_specs=[
            pl.BlockSpec((_BLK, _C), lambda i: (i, 0)),
            anyspec, anyspec, anyspec,
            anyspec, anyspec, anyspec, anyspec, anyspec,
        ],
        out_specs=pl.BlockSpec((_BLK, _C), lambda i: (i, 0)),
        out_shape=jax.ShapeDtypeStruct((_N, _C), jnp.float32),
        scratch_shapes=[
            pltpu.VMEM((3, _C, _C), jnp.float32),
            pltpu.VMEM((5, 1, _C), jnp.float32),
            pltpu.SemaphoreType.DMA((8,)),
        ],
    )(x, W_xz, W_xh, W_lin,
      b_xz.reshape(1, _C), b_hz.reshape(1, _C),
      b_xh.reshape(1, _C), b_hh.reshape(1, _C), b_lin.reshape(1, _C))
